# initial kernel scaffold (unmeasured)
import jax
import jax.numpy as jnp
from jax import lax
from jax.experimental import pallas as pl
from jax.experimental.pallas import tpu as pltpu

N_DEV = 8
C_GLOBAL = 2048
EPS = 1e-5


def kernel(x, t_emb, W_scale, W_shift):
    b, s, c = x.shape

    def body(x_ref, t_ref, wsc_ref, wsh_ref, out_ref, comm_ref, send_sems, recv_sems):
        my_pos = lax.axis_index("i")

        xs = x_ref[...]
        psum = jnp.sum(xs, axis=-1)
        psumsq = jnp.sum(xs * xs, axis=-1)
        stats = jnp.concatenate([psum, psumsq], axis=0)
        comm_ref[my_pos] = stats

        rdmas = []
        for k in range(1, N_DEV):
            peer = lax.rem(my_pos + k, N_DEV)
            rdma = pltpu.make_async_remote_copy(
                src_ref=comm_ref.at[my_pos],
                dst_ref=comm_ref.at[my_pos],
                send_sem=send_sems.at[k],
                recv_sem=recv_sems.at[k],
                device_id=(peer,),
                device_id_type=pl.DeviceIdType.MESH,
            )
            rdma.start()
            rdmas.append(rdma)

        scale = jnp.dot(t_ref[...], wsc_ref[...], preferred_element_type=jnp.float32)
        shift = jnp.dot(t_ref[...], wsh_ref[...], preferred_element_type=jnp.float32)

        for rdma in rdmas:
            rdma.wait()

        total = jnp.sum(comm_ref[...], axis=0)
        mean = total[:b] * (1.0 / C_GLOBAL)
        ex2 = total[b:] * (1.0 / C_GLOBAL)
        var = ex2 - mean * mean
        rstd = lax.rsqrt(var + EPS)

        h = (xs - mean[:, :, None]) * rstd[:, :, None]
        out_ref[...] = h * (1.0 + scale[:, None, :]) + shift[:, None, :]

    return pl.pallas_call(
        body,
        out_shape=jax.ShapeDtypeStruct((b, s, c), jnp.float32),
        in_specs=[pl.BlockSpec(memory_space=pltpu.VMEM)] * 4,
        out_specs=pl.BlockSpec(memory_space=pltpu.VMEM),
        scratch_shapes=[
            pltpu.VMEM((N_DEV, 2 * b, s), jnp.float32),
            pltpu.SemaphoreType.DMA((N_DEV,)),
            pltpu.SemaphoreType.DMA((N_DEV,)),
        ],
        compiler_params=pltpu.CompilerParams(collective_id=0),
    )(x, t_emb, W_scale, W_shift)


# baseline (device time: 20556 ns/iter reference)
import jax
import jax.numpy as jnp
from jax import lax
from jax.experimental import pallas as pl
from jax.experimental.pallas import tpu as pltpu

N_DEV = 8
C_GLOBAL = 2048
EPS = 1e-5


def kernel(x, t_emb, W_scale, W_shift):
    b, s, c = x.shape

    def body(x_ref, t_ref, wsc_ref, wsh_ref, out_ref, comm_ref, send_sems, recv_sems):
        my_pos = lax.axis_index("i")

        xs = x_ref[...]
        psum = jnp.sum(xs, axis=-1)
        psumsq = jnp.sum(xs * xs, axis=-1)
        stats = jnp.concatenate([psum, psumsq], axis=0)
        comm_ref[my_pos] = stats

        rdmas = []
        for k in range(1, N_DEV):
            peer = lax.rem(my_pos + k, N_DEV)
            rdma = pltpu.make_async_remote_copy(
                src_ref=comm_ref.at[my_pos],
                dst_ref=comm_ref.at[my_pos],
                send_sem=send_sems.at[k],
                recv_sem=recv_sems.at[k],
                device_id=(peer,),
                device_id_type=pl.DeviceIdType.MESH,
            )
            rdma.start()
            rdmas.append(rdma)

        scale = jnp.dot(t_ref[...], wsc_ref[...], preferred_element_type=jnp.float32)
        shift = jnp.dot(t_ref[...], wsh_ref[...], preferred_element_type=jnp.float32)

        for rdma in rdmas:
            rdma.wait()

        total = jnp.sum(comm_ref[...], axis=0)
        mean = total[:b] * (1.0 / C_GLOBAL)
        ex2 = total[b:] * (1.0 / C_GLOBAL)
        var = ex2 - mean * mean
        rstd = lax.rsqrt(var + EPS)

        h = (xs - mean[:, :, None]) * rstd[:, :, None]
        out_ref[...] = h * (1.0 + scale[:, None, :]) + shift[:, None, :]

    return pl.pallas_call(
        body,
        out_shape=jax.ShapeDtypeStruct((b, s, c), jnp.float32),
        in_specs=[pl.BlockSpec(memory_space=pltpu.VMEM)] * 4,
        out_specs=pl.BlockSpec(memory_space=pltpu.VMEM),
        scratch_shapes=[
            pltpu.VMEM((N_DEV, 2 * b, s), jnp.float32),
            pltpu.SemaphoreType.DMA((N_DEV,)),
            pltpu.SemaphoreType.DMA((N_DEV,)),
        ],
    )(x, t_emb, W_scale, W_shift)
